# baseline (speedup 1.0000x reference)
import functools

import jax
import jax.numpy as jnp
from jax import lax
from jax.experimental import pallas as pl
from jax.experimental.pallas import tpu as pltpu
from jax.experimental.pallas import tpu_sc as plsc

_N, _T, _H, _W, _C = 4, 2, 132, 132, 64
_TIS = (120, 60, 30)
_PRE = 120


def _sc_body(idx_hbm, g0_hbm, g1_hbm, g2_hbm, out_hbm,
             idx_v, g0_v, g1_v, g2_v, f_v):
    wid = lax.axis_index("s") * 2 + lax.axis_index("c")
    n = wid // 8
    t = (wid // 4) % 2
    kp = wid % 4

    pltpu.sync_copy(idx_hbm, idx_v)
    pltpu.sync_copy(g0_hbm, g0_v)
    pltpu.sync_copy(g1_hbm, g1_v)
    pltpu.sync_copy(g2_hbm, g2_v)

    lanes = lax.iota(jnp.int32, 16)
    nvec = jnp.full((16,), n * 3, jnp.int32)
    idx_t = plsc.load_gather(idx_v, [nvec])
    pre = idx_t * 2 + t
    coor = (pre.astype(jnp.float32) + 0.5) / _PRE * 2.0 - 1.0
    col = kp * 32 + lanes

    for i, (g_v, ti) in enumerate(((g0_v, 120), (g1_v, 60), (g2_v, 30))):
        iz = (coor + 1.0) * 0.5 * (ti - 1)
        z0 = iz.astype(jnp.int32)
        fz = iz - z0.astype(jnp.float32)
        valid1 = (z0 + 1 < ti).astype(jnp.float32)
        z1 = jnp.minimum(z0 + 1, ti - 1)
        w1 = fz * valid1
        w0 = 1.0 - fz
        for hh in (0, 1):
            cvec = col + hh * 16
            a = plsc.load_gather(g_v, [z0, cvec])
            b = plsc.load_gather(g_v, [z1, cvec])
            f_v[pl.ds(i * 32 + hh * 16, 16)] = a * w0 + b * w1

    pltpu.sync_copy(f_v, out_hbm.at[wid])


def _sc_features(idx_flat, g0, g1, g2):
    mesh = plsc.VectorSubcoreMesh(core_axis_name="c", subcore_axis_name="s")
    k = functools.partial(
        pl.kernel,
        out_type=jax.ShapeDtypeStruct((32, 96), jnp.float32),
        mesh=mesh,
        scratch_types=[
            pltpu.VMEM((16,), jnp.int32),
            pltpu.VMEM((120, 128), jnp.float32),
            pltpu.VMEM((60, 128), jnp.float32),
            pltpu.VMEM((30, 128), jnp.float32),
            pltpu.VMEM((96,), jnp.float32),
        ],
        compiler_params=pltpu.CompilerParams(needs_layout_passes=False),
    )(_sc_body)
    return k(idx_flat, g0, g1, g2)


def _tc_body(idx_ref, f_ref, w_ref, b_ref, x_ref, o_ref):
    i = pl.program_id(0)
    j = pl.program_id(1)
    n = i // 2
    t = i % 2
    it = idx_ref[n, 0]
    ih = idx_ref[n, 1]
    iw = idx_ref[n, 2]

    f4 = f_ref[0]
    enc4 = lax.dot_general(f4, w_ref[...], (((1,), (1,)), ((), ())),
                           preferred_element_type=jnp.float32) + b_ref[...]
    c0 = jnp.concatenate([enc4[0:1], enc4[1:2]], axis=1)
    c1 = jnp.concatenate([enc4[2:3], enc4[3:4]], axis=1)

    iw2 = lax.broadcasted_iota(jnp.int32, (66, 128), 0)
    il = lax.broadcasted_iota(jnp.int32, (66, 128), 1)
    wfull = iw2 * 2 + (il >= 64).astype(jnp.int32)
    pxw = iw * 128 + wfull - 2
    mw = ((pxw >= 0) & (pxw < 256)).astype(jnp.float32)
    v0 = (mw * c0)[None]
    v1 = (mw * c1)[None]

    ihv = lax.broadcasted_iota(jnp.int32, (66, 1, 1), 0) + j * 66
    pxh = ih * 128 + ihv - 2
    pxt = it * 2 + t
    tvalid = ((pxt >= 0) & (pxt < 120)).astype(jnp.float32)
    mh = ((pxh >= 0) & (pxh < 256)).astype(jnp.float32) * tvalid
    hodd = (ihv % 2) == 1
    sel = jnp.where(hodd, v1, v0)
    o_ref[0] = x_ref[0] + mh * sel


def _tc_add(xr, idx, f, lw, lb):
    grid = (_N * _T, 2)
    return pl.pallas_call(
        _tc_body,
        grid=grid,
        in_specs=[
            pl.BlockSpec(memory_space=pltpu.SMEM),
            pl.BlockSpec((1, 4, 96), lambda i, j: (i, 0, 0)),
            pl.BlockSpec((64, 96), lambda i, j: (0, 0)),
            pl.BlockSpec((1, 64), lambda i, j: (0, 0)),
            pl.BlockSpec((1, 66, 66, 128), lambda i, j: (i, j, 0, 0)),
        ],
        out_specs=pl.BlockSpec((1, 66, 66, 128), lambda i, j: (i, j, 0, 0)),
        out_shape=jax.ShapeDtypeStruct((_N * _T, 132, 66, 128), jnp.float32),
        compiler_params=pltpu.CompilerParams(
            dimension_semantics=("parallel", "parallel")),
    )(idx, f, lw, lb, xr)


def kernel(x, idx, grid0, grid1, grid2, lin_w, lin_b):
    idx_flat = jnp.zeros((16,), jnp.int32).at[:12].set(idx.reshape(12))
    f = _sc_features(idx_flat, grid0, grid1, grid2)
    xr = x.reshape(_N * _T, _H, _W // 2, 128)
    out = _tc_add(xr, idx, f.reshape(_N * _T, 4, 96), lin_w,
                  lin_b.reshape(1, 64))
    return out.reshape(x.shape)


# speedup vs baseline: 1.5039x; 1.5039x over previous
import functools

import jax
import jax.numpy as jnp
from jax import lax
from jax.experimental import pallas as pl
from jax.experimental.pallas import tpu as pltpu
from jax.experimental.pallas import tpu_sc as plsc

_N, _T, _H, _W, _C = 4, 2, 132, 132, 64
_TIS = (120, 60, 30)
_PRE = 120


def _sc_body(idx_hbm, g0_hbm, g1_hbm, g2_hbm, out_hbm,
             idx_v, g0_v, g1_v, g2_v, f_v):
    wid = lax.axis_index("s") * 2 + lax.axis_index("c")
    n = wid // 8
    t = (wid // 4) % 2
    kp = wid % 4

    pltpu.sync_copy(idx_hbm, idx_v)
    pltpu.sync_copy(g0_hbm, g0_v)
    pltpu.sync_copy(g1_hbm, g1_v)
    pltpu.sync_copy(g2_hbm, g2_v)

    lanes = lax.iota(jnp.int32, 16)
    nvec = jnp.full((16,), n * 3, jnp.int32)
    idx_t = plsc.load_gather(idx_v, [nvec])
    pre = idx_t * 2 + t
    coor = (pre.astype(jnp.float32) + 0.5) / _PRE * 2.0 - 1.0
    col = kp * 32 + lanes

    for i, (g_v, ti) in enumerate(((g0_v, 120), (g1_v, 60), (g2_v, 30))):
        iz = (coor + 1.0) * 0.5 * (ti - 1)
        z0 = iz.astype(jnp.int32)
        fz = iz - z0.astype(jnp.float32)
        valid1 = (z0 + 1 < ti).astype(jnp.float32)
        z1 = jnp.minimum(z0 + 1, ti - 1)
        w1 = fz * valid1
        w0 = 1.0 - fz
        for hh in (0, 1):
            cvec = col + hh * 16
            a = plsc.load_gather(g_v, [z0, cvec])
            b = plsc.load_gather(g_v, [z1, cvec])
            f_v[pl.ds(i * 32 + hh * 16, 16)] = a * w0 + b * w1

    pltpu.sync_copy(f_v, out_hbm.at[wid])


def _sc_features(idx_flat, g0, g1, g2):
    mesh = plsc.VectorSubcoreMesh(core_axis_name="c", subcore_axis_name="s")
    k = functools.partial(
        pl.kernel,
        out_type=jax.ShapeDtypeStruct((32, 96), jnp.float32),
        mesh=mesh,
        scratch_types=[
            pltpu.VMEM((16,), jnp.int32),
            pltpu.VMEM((120, 128), jnp.float32),
            pltpu.VMEM((60, 128), jnp.float32),
            pltpu.VMEM((30, 128), jnp.float32),
            pltpu.VMEM((96,), jnp.float32),
        ],
        compiler_params=pltpu.CompilerParams(needs_layout_passes=False),
    )(_sc_body)
    return k(idx_flat, g0, g1, g2)


def _tc_body(idx_ref, f_ref, w_ref, b_ref, x_ref, o_ref):
    i = pl.program_id(0)
    j = pl.program_id(1)
    n = i // 2
    t = i % 2
    it = idx_ref[n, 0]
    ih = idx_ref[n, 1]
    iw = idx_ref[n, 2]

    f4 = f_ref[0]
    enc4 = lax.dot_general(f4, w_ref[...], (((1,), (1,)), ((), ())),
                           preferred_element_type=jnp.float32) + b_ref[...]

    iwv = lax.broadcasted_iota(jnp.int32, (132, 1), 0)
    pxw = iw * 128 + iwv - 2
    mw = ((pxw >= 0) & (pxw < 256)).astype(jnp.float32)
    wodd = (iwv % 2) == 1
    v0 = (mw * jnp.where(wodd, enc4[1][None, :], enc4[0][None, :]))[None]
    v1 = (mw * jnp.where(wodd, enc4[3][None, :], enc4[2][None, :]))[None]

    ihv = lax.broadcasted_iota(jnp.int32, (66, 1, 1), 0) + j * 66
    pxh = ih * 128 + ihv - 2
    pxt = it * 2 + t
    tvalid = ((pxt >= 0) & (pxt < 120)).astype(jnp.float32)
    mh = ((pxh >= 0) & (pxh < 256)).astype(jnp.float32) * tvalid
    hodd = (ihv % 2) == 1
    sel = jnp.where(hodd, v1, v0)
    o_ref[0] = x_ref[0] + mh * sel


def _tc_add(xr, idx, f, lw, lb):
    grid = (_N * _T, 2)
    return pl.pallas_call(
        _tc_body,
        grid=grid,
        in_specs=[
            pl.BlockSpec(memory_space=pltpu.SMEM),
            pl.BlockSpec((1, 4, 96), lambda i, j: (i, 0, 0)),
            pl.BlockSpec((64, 96), lambda i, j: (0, 0)),
            pl.BlockSpec((1, 64), lambda i, j: (0, 0)),
            pl.BlockSpec((1, 66, 132, 64), lambda i, j: (i, j, 0, 0)),
        ],
        out_specs=pl.BlockSpec((1, 66, 132, 64), lambda i, j: (i, j, 0, 0)),
        out_shape=jax.ShapeDtypeStruct((_N * _T, _H, _W, _C), jnp.float32),
        compiler_params=pltpu.CompilerParams(
            dimension_semantics=("parallel", "parallel")),
    )(idx, f, lw, lb, xr)


def kernel(x, idx, grid0, grid1, grid2, lin_w, lin_b):
    idx_flat = jnp.zeros((16,), jnp.int32).at[:12].set(idx.reshape(12))
    f = _sc_features(idx_flat, grid0, grid1, grid2)
    xr = x.reshape(_N * _T, _H, _W, _C)
    out = _tc_add(xr, idx, f.reshape(_N * _T, 4, 96), lin_w,
                  lin_b.reshape(1, 64))
    return out.reshape(x.shape)
